# baseline (speedup 1.0000x reference)
import functools

import jax
import jax.numpy as jnp
from jax.experimental import pallas as pl
from jax.experimental.pallas import tpu as pltpu

_N = 4096
_DIN = 256
_DOUT = 64
_H = 4
_ALPHA = 0.2
_NEG = -9e15

_RB = 256
_CB = 1024
_PB = 512


def _proj_kernel(x_ref, w_ref, a_ref, wh_ref, e1_ref, e2_ref):
    xb = x_ref[...]
    for h in range(_H):
        whh = jnp.dot(xb, w_ref[h], preferred_element_type=jnp.float32)
        wh_ref[:, h * _DOUT:(h + 1) * _DOUT] = whh
        ah = a_ref[h]
        e1_ref[:, h:h + 1] = jnp.dot(whh, ah[:_DOUT],
                                     preferred_element_type=jnp.float32)
        e2_ref[h:h + 1, :] = jax.lax.dot_general(
            ah[_DOUT:], whh, (((0,), (1,)), ((), ())),
            preferred_element_type=jnp.float32)


def _att_kernel(adj_ref, wh_ref, e1_ref, e2_ref, out_ref, m_ref, l_ref, acc_ref,
                *, ncb):
    j = pl.program_id(1)

    @pl.when(j == 0)
    def _init():
        m_ref[...] = jnp.full_like(m_ref, -jnp.inf)
        l_ref[...] = jnp.zeros_like(l_ref)
        acc_ref[...] = jnp.zeros_like(acc_ref)

    mask = adj_ref[...] > 0
    c0 = j * _CB
    for h in range(_H):
        e1h = e1_ref[:, h:h + 1]
        e2h = e2_ref[h:h + 1, pl.ds(c0, _CB)]
        s = e1h + e2h
        s = jnp.where(s > 0, s, _ALPHA * s)
        s = jnp.where(mask, s, _NEG)
        m_old = m_ref[:, h:h + 1]
        m_new = jnp.maximum(m_old, jnp.max(s, axis=1, keepdims=True))
        p = jnp.exp(s - m_new)
        corr = jnp.exp(m_old - m_new)
        l_ref[:, h:h + 1] = corr * l_ref[:, h:h + 1] + jnp.sum(
            p, axis=1, keepdims=True)
        whj = wh_ref[pl.ds(c0, _CB), h * _DOUT:(h + 1) * _DOUT]
        acc_ref[:, h * _DOUT:(h + 1) * _DOUT] = (
            corr * acc_ref[:, h * _DOUT:(h + 1) * _DOUT]
            + jnp.dot(p, whj, preferred_element_type=jnp.float32))
        m_ref[:, h:h + 1] = m_new

    @pl.when(j == ncb - 1)
    def _fini():
        for h in range(_H):
            out_ref[:, h * _DOUT:(h + 1) * _DOUT] = (
                acc_ref[:, h * _DOUT:(h + 1) * _DOUT] / l_ref[:, h:h + 1])


def kernel(x, adj, W0, a0, W1, a1, W2, a2, W3, a3):
    ws = jnp.stack([W0, W1, W2, W3])
    avs = jnp.stack([a0, a1, a2, a3])

    npb = _N // _PB
    wh, e1, e2 = pl.pallas_call(
        _proj_kernel,
        grid=(npb,),
        in_specs=[
            pl.BlockSpec((_PB, _DIN), lambda i: (i, 0)),
            pl.BlockSpec((_H, _DIN, _DOUT), lambda i: (0, 0, 0)),
            pl.BlockSpec((_H, 2 * _DOUT, 1), lambda i: (0, 0, 0)),
        ],
        out_specs=[
            pl.BlockSpec((_PB, _H * _DOUT), lambda i: (i, 0)),
            pl.BlockSpec((_PB, 8), lambda i: (i, 0)),
            pl.BlockSpec((8, _PB), lambda i: (0, i)),
        ],
        out_shape=[
            jax.ShapeDtypeStruct((_N, _H * _DOUT), jnp.float32),
            jax.ShapeDtypeStruct((_N, 8), jnp.float32),
            jax.ShapeDtypeStruct((8, _N), jnp.float32),
        ],
        compiler_params=pltpu.CompilerParams(
            dimension_semantics=("parallel",)),
    )(x, ws, avs)

    nrb = _N // _RB
    ncb = _N // _CB
    out = pl.pallas_call(
        functools.partial(_att_kernel, ncb=ncb),
        grid=(nrb, ncb),
        in_specs=[
            pl.BlockSpec((_RB, _CB), lambda i, j: (i, j)),
            pl.BlockSpec((_N, _H * _DOUT), lambda i, j: (0, 0)),
            pl.BlockSpec((_RB, 8), lambda i, j: (i, 0)),
            pl.BlockSpec((8, _N), lambda i, j: (0, 0)),
        ],
        out_specs=pl.BlockSpec((_RB, _H * _DOUT), lambda i, j: (i, 0)),
        out_shape=jax.ShapeDtypeStruct((_N, _H * _DOUT), jnp.float32),
        scratch_shapes=[
            pltpu.VMEM((_RB, 8), jnp.float32),
            pltpu.VMEM((_RB, 8), jnp.float32),
            pltpu.VMEM((_RB, _H * _DOUT), jnp.float32),
        ],
        compiler_params=pltpu.CompilerParams(
            dimension_semantics=("parallel", "arbitrary")),
    )(adj, wh, e1, e2)
    return out


# speedup vs baseline: 2.8957x; 2.8957x over previous
import functools

import jax
import jax.numpy as jnp
from jax.experimental import pallas as pl
from jax.experimental.pallas import tpu as pltpu

_N = 4096
_DIN = 256
_DOUT = 64
_H = 4
_ALPHA = 0.2
_EXT = 128

_RB = 256
_CB = 1024
_PB = 512


def _proj_kernel(x_ref, w_ref, a_ref, whext_ref, e1_ref, e2_ref):
    xb = x_ref[...]
    ones = jnp.ones((_PB, 1), jnp.bfloat16)
    zpad = jnp.zeros((_PB, _EXT - _DOUT - 1), jnp.bfloat16)
    for h in range(_H):
        whh = jnp.dot(xb, w_ref[h], preferred_element_type=jnp.float32)
        whext_ref[:, h * _EXT:(h + 1) * _EXT] = jnp.concatenate(
            [whh.astype(jnp.bfloat16), ones, zpad], axis=1)
        ah = a_ref[h]
        e1_ref[:, h:h + 1] = jnp.dot(whh, ah[:_DOUT],
                                     preferred_element_type=jnp.float32)
        e2_ref[h:h + 1, :] = jax.lax.dot_general(
            ah[_DOUT:], whh, (((0,), (1,)), ((), ())),
            preferred_element_type=jnp.float32)


def _att_kernel(adj_ref, whext_ref, e1_ref, e2_ref, out_ref, m_ref, acc_ref,
                *, ncb):
    j = pl.program_id(1)

    @pl.when(j == 0)
    def _init():
        acc_ref[...] = jnp.zeros_like(acc_ref)
        for h in range(_H):
            gm = jnp.max(e2_ref[h:h + 1, :])
            z = e1_ref[:, h:h + 1] + gm
            m_ref[:, h:h + 1] = jnp.maximum(z, _ALPHA * z)

    adjf = adj_ref[...].astype(jnp.float32)
    c0 = j * _CB
    for h in range(_H):
        e1h = e1_ref[:, h:h + 1]
        e2h = e2_ref[h:h + 1, pl.ds(c0, _CB)]
        s = e1h + e2h
        s = jnp.maximum(s, _ALPHA * s)
        p = jnp.exp(s - m_ref[:, h:h + 1]) * adjf
        whj = whext_ref[pl.ds(c0, _CB), h * _EXT:(h + 1) * _EXT]
        acc_ref[:, h * _EXT:(h + 1) * _EXT] += jnp.dot(
            p.astype(jnp.bfloat16), whj, preferred_element_type=jnp.float32)

    @pl.when(j == ncb - 1)
    def _fini():
        for h in range(_H):
            out_ref[:, h * _DOUT:(h + 1) * _DOUT] = (
                acc_ref[:, h * _EXT:h * _EXT + _DOUT]
                / acc_ref[:, h * _EXT + _DOUT:h * _EXT + _DOUT + 1])


def kernel(x, adj, W0, a0, W1, a1, W2, a2, W3, a3):
    ws = jnp.stack([W0, W1, W2, W3])
    avs = jnp.stack([a0, a1, a2, a3])

    npb = _N // _PB
    whext, e1, e2 = pl.pallas_call(
        _proj_kernel,
        grid=(npb,),
        in_specs=[
            pl.BlockSpec((_PB, _DIN), lambda i: (i, 0)),
            pl.BlockSpec((_H, _DIN, _DOUT), lambda i: (0, 0, 0)),
            pl.BlockSpec((_H, 2 * _DOUT, 1), lambda i: (0, 0, 0)),
        ],
        out_specs=[
            pl.BlockSpec((_PB, _H * _EXT), lambda i: (i, 0)),
            pl.BlockSpec((_PB, 8), lambda i: (i, 0)),
            pl.BlockSpec((8, _PB), lambda i: (0, i)),
        ],
        out_shape=[
            jax.ShapeDtypeStruct((_N, _H * _EXT), jnp.bfloat16),
            jax.ShapeDtypeStruct((_N, 8), jnp.float32),
            jax.ShapeDtypeStruct((8, _N), jnp.float32),
        ],
        compiler_params=pltpu.CompilerParams(
            dimension_semantics=("parallel",)),
    )(x, ws, avs)

    nrb = _N // _RB
    ncb = _N // _CB
    out = pl.pallas_call(
        functools.partial(_att_kernel, ncb=ncb),
        grid=(nrb, ncb),
        in_specs=[
            pl.BlockSpec((_RB, _CB), lambda i, j: (i, j)),
            pl.BlockSpec((_N, _H * _EXT), lambda i, j: (0, 0)),
            pl.BlockSpec((_RB, 8), lambda i, j: (i, 0)),
            pl.BlockSpec((8, _N), lambda i, j: (0, 0)),
        ],
        out_specs=pl.BlockSpec((_RB, _H * _DOUT), lambda i, j: (i, 0)),
        out_shape=jax.ShapeDtypeStruct((_N, _H * _DOUT), jnp.float32),
        scratch_shapes=[
            pltpu.VMEM((_RB, 8), jnp.float32),
            pltpu.VMEM((_RB, _H * _EXT), jnp.float32),
        ],
        compiler_params=pltpu.CompilerParams(
            dimension_semantics=("parallel", "arbitrary")),
    )(adj, whext, e1, e2)
    return out


# speedup vs baseline: 3.2825x; 1.1336x over previous
import functools

import jax
import jax.numpy as jnp
import numpy as np
from jax.experimental import pallas as pl
from jax.experimental.pallas import tpu as pltpu

_N = 4096
_DIN = 256
_DOUT = 64
_H = 4
_ALPHA = 0.2
_EXT = 128
_LOG2E = float(np.log2(np.e))

_RB = 256
_CB = 1024
_PB = 512


def _proj_kernel(x_ref, w_ref, a_ref, whext_ref, e1_ref, e2_ref, e2s_ref):
    xb = x_ref[...]
    ones = jnp.ones((_PB, 1), jnp.bfloat16)
    zpad = jnp.zeros((_PB, _EXT - _DOUT - 1), jnp.bfloat16)
    for h in range(_H):
        whh = jnp.dot(xb, w_ref[h], preferred_element_type=jnp.float32)
        whext_ref[:, h * _EXT:(h + 1) * _EXT] = jnp.concatenate(
            [whh.astype(jnp.bfloat16), ones, zpad], axis=1)
        ah = a_ref[h]
        e1_ref[:, h:h + 1] = _LOG2E * jnp.dot(
            whh, ah[:_DOUT], preferred_element_type=jnp.float32)
        e2l = _LOG2E * jax.lax.dot_general(
            ah[_DOUT:], whh, (((0,), (1,)), ((), ())),
            preferred_element_type=jnp.float32)
        e2_ref[h:h + 1, :] = e2l
        e2s_ref[h:h + 1, :] = _ALPHA * e2l


def _att_kernel(adj_ref, whext_ref, e1_ref, e2_ref, e2s_ref, out_ref,
                b1_ref, b2_ref, acc_ref, *, ncb):
    j = pl.program_id(1)

    @pl.when(j == 0)
    def _init():
        acc_ref[...] = jnp.zeros_like(acc_ref)
        for h in range(_H):
            gm = jnp.max(e2_ref[h:h + 1, :])
            e1h = e1_ref[:, h:h + 1]
            z = e1h + gm
            ml = jnp.maximum(z, _ALPHA * z)
            b1_ref[:, h:h + 1] = e1h - ml
            b2_ref[:, h:h + 1] = _ALPHA * e1h - ml

    adjf = adj_ref[...].astype(jnp.float32)
    c0 = j * _CB
    for h in range(_H):
        b1 = b1_ref[:, h:h + 1]
        b2 = b2_ref[:, h:h + 1]
        e2h = e2_ref[h:h + 1, pl.ds(c0, _CB)]
        e2sh = e2s_ref[h:h + 1, pl.ds(c0, _CB)]
        arg = jnp.maximum(b1 + e2h, b2 + e2sh)
        p = jnp.exp2(arg) * adjf
        whj = whext_ref[pl.ds(c0, _CB), h * _EXT:(h + 1) * _EXT]
        acc_ref[:, h * _EXT:(h + 1) * _EXT] += jnp.dot(
            p.astype(jnp.bfloat16), whj, preferred_element_type=jnp.float32)

    @pl.when(j == ncb - 1)
    def _fini():
        for h in range(_H):
            out_ref[:, h * _DOUT:(h + 1) * _DOUT] = (
                acc_ref[:, h * _EXT:h * _EXT + _DOUT]
                / acc_ref[:, h * _EXT + _DOUT:h * _EXT + _DOUT + 1])


def kernel(x, adj, W0, a0, W1, a1, W2, a2, W3, a3):
    ws = jnp.stack([W0, W1, W2, W3])
    avs = jnp.stack([a0, a1, a2, a3])

    npb = _N // _PB
    whext, e1, e2, e2s = pl.pallas_call(
        _proj_kernel,
        grid=(npb,),
        in_specs=[
            pl.BlockSpec((_PB, _DIN), lambda i: (i, 0)),
            pl.BlockSpec((_H, _DIN, _DOUT), lambda i: (0, 0, 0)),
            pl.BlockSpec((_H, 2 * _DOUT, 1), lambda i: (0, 0, 0)),
        ],
        out_specs=[
            pl.BlockSpec((_PB, _H * _EXT), lambda i: (i, 0)),
            pl.BlockSpec((_PB, 8), lambda i: (i, 0)),
            pl.BlockSpec((8, _PB), lambda i: (0, i)),
            pl.BlockSpec((8, _PB), lambda i: (0, i)),
        ],
        out_shape=[
            jax.ShapeDtypeStruct((_N, _H * _EXT), jnp.bfloat16),
            jax.ShapeDtypeStruct((_N, 8), jnp.float32),
            jax.ShapeDtypeStruct((8, _N), jnp.float32),
            jax.ShapeDtypeStruct((8, _N), jnp.float32),
        ],
        compiler_params=pltpu.CompilerParams(
            dimension_semantics=("parallel",)),
    )(x, ws, avs)

    nrb = _N // _RB
    ncb = _N // _CB
    out = pl.pallas_call(
        functools.partial(_att_kernel, ncb=ncb),
        grid=(nrb, ncb),
        in_specs=[
            pl.BlockSpec((_RB, _CB), lambda i, j: (i, j)),
            pl.BlockSpec((_N, _H * _EXT), lambda i, j: (0, 0)),
            pl.BlockSpec((_RB, 8), lambda i, j: (i, 0)),
            pl.BlockSpec((8, _N), lambda i, j: (0, 0)),
            pl.BlockSpec((8, _N), lambda i, j: (0, 0)),
        ],
        out_specs=pl.BlockSpec((_RB, _H * _DOUT), lambda i, j: (i, 0)),
        out_shape=jax.ShapeDtypeStruct((_N, _H * _DOUT), jnp.float32),
        scratch_shapes=[
            pltpu.VMEM((_RB, 8), jnp.float32),
            pltpu.VMEM((_RB, 8), jnp.float32),
            pltpu.VMEM((_RB, _H * _EXT), jnp.float32),
        ],
        compiler_params=pltpu.CompilerParams(
            dimension_semantics=("parallel", "arbitrary")),
    )(adj, whext, e1, e2, e2s)
    return out


# speedup vs baseline: 4.9017x; 1.4933x over previous
import jax
import jax.numpy as jnp
import numpy as np
from jax.experimental import pallas as pl
from jax.experimental.pallas import tpu as pltpu

_N = 4096
_DIN = 256
_DOUT = 64
_H = 4
_ALPHA = 0.2
_EXT = 128
_LOG2E = float(np.log2(np.e))

_RB = 256
_PB = 512


def _proj_kernel(x_ref, w_ref, a_ref, whext_ref, e1_ref, e2_ref, e2s_ref):
    xb = x_ref[...]
    ones = jnp.ones((_PB, 1), jnp.bfloat16)
    zpad = jnp.zeros((_PB, _EXT - _DOUT - 1), jnp.bfloat16)
    for h in range(_H):
        whh = jnp.dot(xb, w_ref[h], preferred_element_type=jnp.float32)
        whext_ref[:, h * _EXT:(h + 1) * _EXT] = jnp.concatenate(
            [whh.astype(jnp.bfloat16), ones, zpad], axis=1)
        ah = a_ref[h]
        e1_ref[:, h:h + 1] = _LOG2E * jnp.dot(
            whh, ah[:_DOUT], preferred_element_type=jnp.float32)
        e2l = _LOG2E * jax.lax.dot_general(
            ah[_DOUT:], whh, (((0,), (1,)), ((), ())),
            preferred_element_type=jnp.float32)
        e2_ref[h:h + 1, :] = e2l.astype(jnp.bfloat16)
        e2s_ref[h:h + 1, :] = (_ALPHA * e2l).astype(jnp.bfloat16)


def _att_kernel(adj_ref, whext_ref, e1_ref, e2_ref, e2s_ref, out_ref):
    adjf = adj_ref[...].astype(jnp.bfloat16)
    for h in range(_H):
        gm = jnp.max(e2_ref[h:h + 1, :].astype(jnp.float32))
        e1h = e1_ref[:, h:h + 1]
        z = e1h + gm
        ml = jnp.maximum(z, _ALPHA * z)
        b1 = (e1h - ml).astype(jnp.bfloat16)
        b2 = (_ALPHA * e1h - ml).astype(jnp.bfloat16)
        e2h = e2_ref[h:h + 1, :]
        e2sh = e2s_ref[h:h + 1, :]
        arg = jnp.maximum(b1 + e2h, b2 + e2sh)
        p = jnp.exp2(arg) * adjf
        whj = whext_ref[:, h * _EXT:(h + 1) * _EXT]
        res = jnp.dot(p, whj, preferred_element_type=jnp.float32)
        out_ref[:, h * _DOUT:(h + 1) * _DOUT] = (
            res[:, :_DOUT] / res[:, _DOUT:_DOUT + 1])


def kernel(x, adj, W0, a0, W1, a1, W2, a2, W3, a3):
    ws = jnp.stack([W0, W1, W2, W3])
    avs = jnp.stack([a0, a1, a2, a3])

    npb = _N // _PB
    whext, e1, e2, e2s = pl.pallas_call(
        _proj_kernel,
        grid=(npb,),
        in_specs=[
            pl.BlockSpec((_PB, _DIN), lambda i: (i, 0)),
            pl.BlockSpec((_H, _DIN, _DOUT), lambda i: (0, 0, 0)),
            pl.BlockSpec((_H, 2 * _DOUT, 1), lambda i: (0, 0, 0)),
        ],
        out_specs=[
            pl.BlockSpec((_PB, _H * _EXT), lambda i: (i, 0)),
            pl.BlockSpec((_PB, 8), lambda i: (i, 0)),
            pl.BlockSpec((8, _PB), lambda i: (0, i)),
            pl.BlockSpec((8, _PB), lambda i: (0, i)),
        ],
        out_shape=[
            jax.ShapeDtypeStruct((_N, _H * _EXT), jnp.bfloat16),
            jax.ShapeDtypeStruct((_N, 8), jnp.float32),
            jax.ShapeDtypeStruct((8, _N), jnp.bfloat16),
            jax.ShapeDtypeStruct((8, _N), jnp.bfloat16),
        ],
        compiler_params=pltpu.CompilerParams(
            dimension_semantics=("parallel",)),
    )(x, ws, avs)

    nrb = _N // _RB
    out = pl.pallas_call(
        _att_kernel,
        grid=(nrb,),
        in_specs=[
            pl.BlockSpec((_RB, _N), lambda i: (i, 0)),
            pl.BlockSpec((_N, _H * _EXT), lambda i: (0, 0)),
            pl.BlockSpec((_RB, 8), lambda i: (i, 0)),
            pl.BlockSpec((8, _N), lambda i: (0, 0)),
            pl.BlockSpec((8, _N), lambda i: (0, 0)),
        ],
        out_specs=pl.BlockSpec((_RB, _H * _DOUT), lambda i: (i, 0)),
        out_shape=jax.ShapeDtypeStruct((_N, _H * _DOUT), jnp.float32),
        compiler_params=pltpu.CompilerParams(
            dimension_semantics=("parallel",)),
    )(adj, whext, e1, e2, e2s)
    return out


# speedup vs baseline: 5.3572x; 1.0929x over previous
import jax
import jax.numpy as jnp
import numpy as np
from jax.experimental import pallas as pl
from jax.experimental.pallas import tpu as pltpu

_N = 4096
_DIN = 256
_DOUT = 64
_H = 4
_ALPHA = 0.2
_EXT = 128
_LOG2E = float(np.log2(np.e))

_RB = 512
_PB = 512


def _proj_kernel(x_ref, w_ref, a_ref, whext_ref, e1_ref, e2_ref, e2s_ref):
    xb = x_ref[...]
    ones = jnp.ones((_PB, 1), jnp.bfloat16)
    zpad = jnp.zeros((_PB, _EXT - _DOUT - 1), jnp.bfloat16)
    for h in range(_H):
        whh = jnp.dot(xb, w_ref[h], preferred_element_type=jnp.float32)
        whext_ref[:, h * _EXT:(h + 1) * _EXT] = jnp.concatenate(
            [whh.astype(jnp.bfloat16), ones, zpad], axis=1)
        ah = a_ref[h]
        e1_ref[:, h:h + 1] = _LOG2E * jnp.dot(
            whh, ah[:_DOUT], preferred_element_type=jnp.float32)
        e2l = _LOG2E * jax.lax.dot_general(
            ah[_DOUT:], whh, (((0,), (1,)), ((), ())),
            preferred_element_type=jnp.float32)
        e2_ref[h:h + 1, :] = e2l.astype(jnp.bfloat16)
        e2s_ref[h:h + 1, :] = (_ALPHA * e2l).astype(jnp.bfloat16)


def _att_kernel(adj_ref, whext_ref, e1_ref, e2_ref, e2s_ref, out_ref):
    adjf = adj_ref[...].astype(jnp.bfloat16)
    for h in range(_H):
        gm = jnp.max(e2_ref[h:h + 1, :].astype(jnp.float32))
        e1h = e1_ref[:, h:h + 1]
        z = e1h + gm
        ml = jnp.maximum(z, _ALPHA * z)
        b1 = (e1h - ml).astype(jnp.bfloat16)
        b2 = (_ALPHA * e1h - ml).astype(jnp.bfloat16)
        e2h = e2_ref[h:h + 1, :]
        e2sh = e2s_ref[h:h + 1, :]
        arg = jnp.maximum(b1 + e2h, b2 + e2sh)
        p = jnp.exp2(arg) * adjf
        whj = whext_ref[:, h * _EXT:(h + 1) * _EXT]
        res = jnp.dot(p, whj, preferred_element_type=jnp.float32)
        out_ref[:, h * _DOUT:(h + 1) * _DOUT] = (
            res[:, :_DOUT] / res[:, _DOUT:_DOUT + 1])


def kernel(x, adj, W0, a0, W1, a1, W2, a2, W3, a3):
    ws = jnp.stack([W0, W1, W2, W3])
    avs = jnp.stack([a0, a1, a2, a3])

    npb = _N // _PB
    whext, e1, e2, e2s = pl.pallas_call(
        _proj_kernel,
        grid=(npb,),
        in_specs=[
            pl.BlockSpec((_PB, _DIN), lambda i: (i, 0)),
            pl.BlockSpec((_H, _DIN, _DOUT), lambda i: (0, 0, 0)),
            pl.BlockSpec((_H, 2 * _DOUT, 1), lambda i: (0, 0, 0)),
        ],
        out_specs=[
            pl.BlockSpec((_PB, _H * _EXT), lambda i: (i, 0)),
            pl.BlockSpec((_PB, 8), lambda i: (i, 0)),
            pl.BlockSpec((8, _PB), lambda i: (0, i)),
            pl.BlockSpec((8, _PB), lambda i: (0, i)),
        ],
        out_shape=[
            jax.ShapeDtypeStruct((_N, _H * _EXT), jnp.bfloat16),
            jax.ShapeDtypeStruct((_N, 8), jnp.float32),
            jax.ShapeDtypeStruct((8, _N), jnp.bfloat16),
            jax.ShapeDtypeStruct((8, _N), jnp.bfloat16),
        ],
        compiler_params=pltpu.CompilerParams(
            dimension_semantics=("parallel",)),
    )(x, ws, avs)

    nrb = _N // _RB
    out = pl.pallas_call(
        _att_kernel,
        grid=(nrb,),
        in_specs=[
            pl.BlockSpec((_RB, _N), lambda i: (i, 0)),
            pl.BlockSpec((_N, _H * _EXT), lambda i: (0, 0)),
            pl.BlockSpec((_RB, 8), lambda i: (i, 0)),
            pl.BlockSpec((8, _N), lambda i: (0, 0)),
            pl.BlockSpec((8, _N), lambda i: (0, 0)),
        ],
        out_specs=pl.BlockSpec((_RB, _H * _DOUT), lambda i: (i, 0)),
        out_shape=jax.ShapeDtypeStruct((_N, _H * _DOUT), jnp.float32),
        compiler_params=pltpu.CompilerParams(
            dimension_semantics=("parallel",)),
    )(adj, whext, e1, e2, e2s)
    return out


# speedup vs baseline: 5.7486x; 1.0731x over previous
import jax
import jax.numpy as jnp
import numpy as np
from jax.experimental import pallas as pl
from jax.experimental.pallas import tpu as pltpu

_N = 4096
_DIN = 256
_DOUT = 64
_H = 4
_ALPHA = 0.2
_EXT = 128
_LOG2E = float(np.log2(np.e))

_RB = 512


def _att_kernel(x_ref, adj_ref, wcat_ref, a1_ref, a2_ref, out_ref,
                whext_ref, e2_ref, e2s_ref, b1_ref, b2_ref):
    i = pl.program_id(0)

    @pl.when(i == 0)
    def _prologue():
        wh = jnp.dot(x_ref[...], wcat_ref[...],
                     preferred_element_type=jnp.float32)
        whb = wh.astype(jnp.bfloat16)
        ones = jnp.ones((_N, 1), jnp.bfloat16)
        zpad = jnp.zeros((_N, _EXT - _DOUT - 1), jnp.bfloat16)
        for h in range(_H):
            whext_ref[:, h * _EXT:(h + 1) * _EXT] = jnp.concatenate(
                [whb[:, h * _DOUT:(h + 1) * _DOUT], ones, zpad], axis=1)
        e1 = jnp.dot(wh, a1_ref[...],
                     preferred_element_type=jnp.float32)
        e2 = jax.lax.dot_general(
            a2_ref[...], wh, (((0,), (1,)), ((), ())),
            preferred_element_type=jnp.float32)
        e2_ref[...] = e2.astype(jnp.bfloat16)
        e2s_ref[...] = (_ALPHA * e2).astype(jnp.bfloat16)
        for h in range(_H):
            gm = jnp.max(e2[h:h + 1, :])
            z = e1[:, h:h + 1] + gm
            ml = jnp.maximum(z, _ALPHA * z)
            b1_ref[:, h:h + 1] = (e1[:, h:h + 1] - ml).astype(jnp.bfloat16)
            b2_ref[:, h:h + 1] = (
                _ALPHA * e1[:, h:h + 1] - ml).astype(jnp.bfloat16)

    r0 = i * _RB
    adjf = adj_ref[...].astype(jnp.bfloat16)
    for h in range(_H):
        b1 = b1_ref[pl.ds(r0, _RB), h:h + 1]
        b2 = b2_ref[pl.ds(r0, _RB), h:h + 1]
        e2h = e2_ref[h:h + 1, :]
        e2sh = e2s_ref[h:h + 1, :]
        arg = jnp.maximum(b1 + e2h, b2 + e2sh)
        p = jnp.exp2(arg) * adjf
        whj = whext_ref[:, h * _EXT:(h + 1) * _EXT]
        res = jnp.dot(p, whj, preferred_element_type=jnp.float32)
        out_ref[:, h * _DOUT:(h + 1) * _DOUT] = (
            res[:, :_DOUT] / res[:, _DOUT:_DOUT + 1])


def kernel(x, adj, W0, a0, W1, a1, W2, a2, W3, a3):
    wcat = jnp.concatenate([W0, W1, W2, W3], axis=1)
    a1blk = jnp.zeros((_DIN, 8), jnp.float32)
    a2blk = jnp.zeros((_DIN, 8), jnp.float32)
    for h, ah in enumerate((a0, a1, a2, a3)):
        a1blk = a1blk.at[h * _DOUT:(h + 1) * _DOUT, h].set(
            _LOG2E * ah[:_DOUT, 0])
        a2blk = a2blk.at[h * _DOUT:(h + 1) * _DOUT, h].set(
            _LOG2E * ah[_DOUT:, 0])

    nrb = _N // _RB
    out = pl.pallas_call(
        _att_kernel,
        grid=(nrb,),
        in_specs=[
            pl.BlockSpec((_N, _DIN), lambda i: (0, 0)),
            pl.BlockSpec((_RB, _N), lambda i: (i, 0)),
            pl.BlockSpec((_DIN, _H * _DOUT), lambda i: (0, 0)),
            pl.BlockSpec((_DIN, 8), lambda i: (0, 0)),
            pl.BlockSpec((_DIN, 8), lambda i: (0, 0)),
        ],
        out_specs=pl.BlockSpec((_RB, _H * _DOUT), lambda i: (i, 0)),
        out_shape=jax.ShapeDtypeStruct((_N, _H * _DOUT), jnp.float32),
        scratch_shapes=[
            pltpu.VMEM((_N, _H * _EXT), jnp.bfloat16),
            pltpu.VMEM((8, _N), jnp.bfloat16),
            pltpu.VMEM((8, _N), jnp.bfloat16),
            pltpu.VMEM((_N, 8), jnp.bfloat16),
            pltpu.VMEM((_N, 8), jnp.bfloat16),
        ],
        compiler_params=pltpu.CompilerParams(
            dimension_semantics=("arbitrary",)),
    )(x, adj, wcat, a1blk, a2blk)
    return out


# speedup vs baseline: 6.1405x; 1.0682x over previous
import jax
import jax.numpy as jnp
import numpy as np
from jax.experimental import pallas as pl
from jax.experimental.pallas import tpu as pltpu

_N = 4096
_DIN = 256
_DOUT = 64
_H = 4
_ALPHA = 0.2
_EXT = 128
_LOG2E = float(np.log2(np.e))

_RB = 512


def _att_kernel(x_ref, adj_ref, wcat_ref, a1_ref, a2_ref, out_ref,
                whext_ref, e2_ref, e2s_ref, b1_ref, b2_ref):
    i = pl.program_id(0)

    @pl.when(i == 0)
    def _prologue():
        wh = jnp.dot(x_ref[...], wcat_ref[...],
                     preferred_element_type=jnp.float32)
        whb = wh.astype(jnp.bfloat16)
        ones = jnp.ones((_N, 1), jnp.bfloat16)
        zpad = jnp.zeros((_N, _EXT - _DOUT - 1), jnp.bfloat16)
        for h in range(_H):
            whext_ref[:, h * _EXT:(h + 1) * _EXT] = jnp.concatenate(
                [whb[:, h * _DOUT:(h + 1) * _DOUT], ones, zpad], axis=1)
        e1t = jax.lax.dot_general(
            a1_ref[...], wh, (((0,), (1,)), ((), ())),
            preferred_element_type=jnp.float32)
        e2 = jax.lax.dot_general(
            a2_ref[...], wh, (((0,), (1,)), ((), ())),
            preferred_element_type=jnp.float32)
        e2_ref[...] = e2.astype(jnp.bfloat16)
        e2s_ref[...] = (_ALPHA * e2).astype(jnp.bfloat16)
        gm = jnp.max(e2, axis=1, keepdims=True)
        z = e1t + gm
        ml = jnp.maximum(z, _ALPHA * z)
        b1_ref[...] = (e1t - ml).astype(jnp.bfloat16).T
        b2_ref[...] = (_ALPHA * e1t - ml).astype(jnp.bfloat16).T

    r0 = i * _RB
    adjf = adj_ref[...].astype(jnp.bfloat16)
    for h in range(_H):
        b1 = b1_ref[pl.ds(r0, _RB), h:h + 1]
        b2 = b2_ref[pl.ds(r0, _RB), h:h + 1]
        e2h = e2_ref[h:h + 1, :]
        e2sh = e2s_ref[h:h + 1, :]
        arg = jnp.maximum(b1 + e2h, b2 + e2sh)
        p = jnp.exp2(arg) * adjf
        whj = whext_ref[:, h * _EXT:(h + 1) * _EXT]
        res = jnp.dot(p, whj, preferred_element_type=jnp.float32)
        out_ref[:, h * _DOUT:(h + 1) * _DOUT] = (
            res[:, :_DOUT] / res[:, _DOUT:_DOUT + 1])


def kernel(x, adj, W0, a0, W1, a1, W2, a2, W3, a3):
    wcat = jnp.concatenate([W0, W1, W2, W3], axis=1)
    a1blk = jnp.zeros((_DIN, 8), jnp.float32)
    a2blk = jnp.zeros((_DIN, 8), jnp.float32)
    for h, ah in enumerate((a0, a1, a2, a3)):
        a1blk = a1blk.at[h * _DOUT:(h + 1) * _DOUT, h].set(
            _LOG2E * ah[:_DOUT, 0])
        a2blk = a2blk.at[h * _DOUT:(h + 1) * _DOUT, h].set(
            _LOG2E * ah[_DOUT:, 0])

    nrb = _N // _RB
    out = pl.pallas_call(
        _att_kernel,
        grid=(nrb,),
        in_specs=[
            pl.BlockSpec((_N, _DIN), lambda i: (0, 0)),
            pl.BlockSpec((_RB, _N), lambda i: (i, 0)),
            pl.BlockSpec((_DIN, _H * _DOUT), lambda i: (0, 0)),
            pl.BlockSpec((_DIN, 8), lambda i: (0, 0)),
            pl.BlockSpec((_DIN, 8), lambda i: (0, 0)),
        ],
        out_specs=pl.BlockSpec((_RB, _H * _DOUT), lambda i: (i, 0)),
        out_shape=jax.ShapeDtypeStruct((_N, _H * _DOUT), jnp.float32),
        scratch_shapes=[
            pltpu.VMEM((_N, _H * _EXT), jnp.bfloat16),
            pltpu.VMEM((8, _N), jnp.bfloat16),
            pltpu.VMEM((8, _N), jnp.bfloat16),
            pltpu.VMEM((_N, 8), jnp.bfloat16),
            pltpu.VMEM((_N, 8), jnp.bfloat16),
        ],
        compiler_params=pltpu.CompilerParams(
            dimension_semantics=("arbitrary",)),
    )(x, adj, wcat, a1blk, a2blk)
    return out
